# Initial kernel scaffold; baseline (speedup 1.0000x reference)
import functools

import jax
import jax.numpy as jnp
from jax import lax
from jax.experimental import pallas as pl
from jax.experimental.pallas import tpu as pltpu
from jax.experimental.pallas import tpu_sc as plsc

N = 10000
E = 320000
D_IN = 128
D_OUT = 256

NPAD = 10240
NC, NS = 2, 16
NW = NC * NS
CHUNK = 128
E2 = E + N
CPT = 81
EPT = CPT * CHUNK
E_PAD = NW * EPT
STRIPE = NPAD // NS
BM = 512
NB = NPAD // BM


def _zero_vmem_2d(ref, nrow, ncol):
    @pl.loop(0, nrow)
    def _(j):
        for g in range(ncol // 16):
            ref[j, pl.ds(g * 16, 16)] = jnp.zeros((16,), jnp.float32)


@functools.cache
def _sc_calls():
    mesh = plsc.VectorSubcoreMesh(
        core_axis_name="c", subcore_axis_name="s", num_cores=NC, num_subcores=NS
    )

    @functools.partial(
        pl.kernel,
        out_type=jax.ShapeDtypeStruct((NC, NPAD), jnp.float32),
        mesh=mesh,
        scratch_types=[
            pltpu.VMEM((CHUNK,), jnp.int32),
            pltpu.VMEM((CHUNK,), jnp.float32),
            pltpu.VMEM((STRIPE,), jnp.float32),
            pltpu.VMEM_SHARED((NPAD,), jnp.float32),
        ],
    )
    def deg_call(dst_hbm, ew_hbm, deg_hbm, dstv, ewv, degv, deg_sh):
        c = lax.axis_index("c")
        s = lax.axis_index("s")
        wid = c * NS + s

        @pl.loop(0, STRIPE // 16)
        def _(i):
            degv[pl.ds(i * 16, 16)] = jnp.zeros((16,), jnp.float32)

        pltpu.sync_copy(degv, deg_sh.at[pl.ds(s * STRIPE, STRIPE)])
        plsc.subcore_barrier()

        base = wid * EPT

        @pl.loop(0, CPT)
        def _(k):
            off = pl.multiple_of(base + k * CHUNK, 8)
            pltpu.sync_copy(dst_hbm.at[pl.ds(off, CHUNK)], dstv)
            pltpu.sync_copy(ew_hbm.at[pl.ds(off, CHUNK)], ewv)
            pltpu.sync_copy(ewv, deg_sh.at[dstv], add=True)

        plsc.subcore_barrier()
        pltpu.sync_copy(deg_sh.at[pl.ds(s * STRIPE, STRIPE)], degv)
        pltpu.sync_copy(degv, deg_hbm.at[c, pl.ds(s * STRIPE, STRIPE)])

    @functools.partial(
        pl.kernel,
        out_type=jax.ShapeDtypeStruct((NC, NPAD, D_IN), jnp.float32),
        mesh=mesh,
        scratch_types=[
            pltpu.VMEM((NPAD,), jnp.float32),
            pltpu.VMEM((CHUNK,), jnp.int32),
            pltpu.VMEM((CHUNK,), jnp.int32),
            pltpu.VMEM((CHUNK,), jnp.float32),
            pltpu.VMEM((CHUNK,), jnp.float32),
            pltpu.VMEM((CHUNK, D_IN), jnp.float32),
            pltpu.SemaphoreType.DMA,
            pltpu.VMEM_SHARED((NPAD, D_IN), jnp.float32),
        ],
    )
    def agg_call(src_hbm, dst_hbm, ew_hbm, dis_hbm, x_hbm, agg_hbm,
                 disv, srcv, dstv, ewv, nrmv, rows, sem, acc_sh):
        c = lax.axis_index("c")
        s = lax.axis_index("s")
        wid = c * NS + s

        _zero_vmem_2d(rows, CHUNK, D_IN)
        for t in range(STRIPE // CHUNK):
            pltpu.sync_copy(
                rows, acc_sh.at[pl.ds(s * STRIPE + t * CHUNK, CHUNK), :]
            )
        pltpu.sync_copy(dis_hbm, disv)
        plsc.subcore_barrier()

        base = wid * EPT

        @pl.loop(0, CPT)
        def _(k):
            off = pl.multiple_of(base + k * CHUNK, 8)
            pltpu.sync_copy(src_hbm.at[pl.ds(off, CHUNK)], srcv)
            pltpu.sync_copy(dst_hbm.at[pl.ds(off, CHUNK)], dstv)
            pltpu.sync_copy(ew_hbm.at[pl.ds(off, CHUNK)], ewv)
            pltpu.async_copy(x_hbm.at[srcv], rows, sem).wait()
            for g in range(CHUNK // 16):
                sv = srcv[pl.ds(g * 16, 16)]
                dv = dstv[pl.ds(g * 16, 16)]
                dis_s = plsc.load_gather(disv, [sv])
                dis_d = plsc.load_gather(disv, [dv])
                nrmv[pl.ds(g * 16, 16)] = dis_s * ewv[pl.ds(g * 16, 16)] * dis_d

            @pl.loop(0, CHUNK)
            def _(j):
                nj = nrmv[j]
                for g in range(D_IN // 16):
                    rows[j, pl.ds(g * 16, 16)] = rows[j, pl.ds(g * 16, 16)] * nj

            pltpu.sync_copy(rows, acc_sh.at[dstv], add=True)

        plsc.subcore_barrier()
        for t in range(STRIPE // CHUNK):
            pltpu.sync_copy(
                acc_sh.at[pl.ds(s * STRIPE + t * CHUNK, CHUNK), :], rows
            )
            pltpu.sync_copy(
                rows, agg_hbm.at[c, pl.ds(s * STRIPE + t * CHUNK, CHUNK), :]
            )

    return deg_call, agg_call


def _dis_body(degp_ref, dis_ref):
    d = degp_ref[0] + degp_ref[1]
    dis_ref[...] = jnp.where(d > 0, lax.rsqrt(d), 0.0)


def _mm_body(agg_ref, w_ref, b_ref, out_ref, st_ref, s1, s2):
    i = pl.program_id(0)

    @pl.when(i == 0)
    def _():
        s1[...] = jnp.zeros_like(s1)
        s2[...] = jnp.zeros_like(s2)

    a = agg_ref[0] + agg_ref[1]
    o = jnp.dot(a, w_ref[...], preferred_element_type=jnp.float32) + b_ref[...]
    out_ref[...] = o
    rid = i * BM + lax.broadcasted_iota(jnp.int32, (BM, 1), 0)
    m = (rid < N).astype(jnp.float32)
    om = o * m
    s1[...] += jnp.sum(om, axis=0, keepdims=True)
    s2[...] += jnp.sum(om * o, axis=0, keepdims=True)

    @pl.when(i == pl.num_programs(0) - 1)
    def _():
        mean = s1[...] / N
        var = s2[...] / N - (s1[...] / N) * (s1[...] / N)
        st_ref[0:1, :] = mean
        st_ref[1:2, :] = var


def _bn_body(o_ref, st_ref, g_ref, be_ref, out_ref):
    mean = st_ref[0:1, :]
    var = st_ref[1:2, :]
    scale = lax.rsqrt(var + 1e-5) * g_ref[...]
    out_ref[...] = (o_ref[...] - mean) * scale + be_ref[...]


def kernel(x, edge_index, edge_weight, W, b, gamma, beta):
    src = edge_index[0].astype(jnp.int32)
    dst = edge_index[1].astype(jnp.int32)
    ew = edge_weight.astype(jnp.float32)
    loop_idx = jnp.arange(N, dtype=jnp.int32)
    npad_e = E_PAD - E2
    src2 = jnp.concatenate([src, loop_idx, jnp.zeros((npad_e,), jnp.int32)])
    dst2 = jnp.concatenate([dst, loop_idx, jnp.zeros((npad_e,), jnp.int32)])
    ew2 = jnp.concatenate(
        [ew, jnp.ones((N,), jnp.float32), jnp.zeros((npad_e,), jnp.float32)]
    )
    x_pad = jnp.pad(x, ((0, NPAD - N), (0, 0)))

    deg_call, agg_call = _sc_calls()
    deg_p = deg_call(dst2, ew2)

    dis = pl.pallas_call(
        _dis_body,
        out_shape=jax.ShapeDtypeStruct((NPAD // 128, 128), jnp.float32),
    )(deg_p.reshape(NC, NPAD // 128, 128)).reshape(NPAD)

    agg_p = agg_call(src2, dst2, ew2, dis, x_pad)

    out_raw, stats = pl.pallas_call(
        _mm_body,
        grid=(NB,),
        in_specs=[
            pl.BlockSpec((NC, BM, D_IN), lambda i: (0, i, 0)),
            pl.BlockSpec((D_IN, D_OUT), lambda i: (0, 0)),
            pl.BlockSpec((1, D_OUT), lambda i: (0, 0)),
        ],
        out_specs=[
            pl.BlockSpec((BM, D_OUT), lambda i: (i, 0)),
            pl.BlockSpec((2, D_OUT), lambda i: (0, 0)),
        ],
        out_shape=[
            jax.ShapeDtypeStruct((NPAD, D_OUT), jnp.float32),
            jax.ShapeDtypeStruct((2, D_OUT), jnp.float32),
        ],
        scratch_shapes=[
            pltpu.VMEM((1, D_OUT), jnp.float32),
            pltpu.VMEM((1, D_OUT), jnp.float32),
        ],
    )(agg_p, W, b.reshape(1, D_OUT))

    out = pl.pallas_call(
        _bn_body,
        grid=(NB,),
        in_specs=[
            pl.BlockSpec((BM, D_OUT), lambda i: (i, 0)),
            pl.BlockSpec((2, D_OUT), lambda i: (0, 0)),
            pl.BlockSpec((1, D_OUT), lambda i: (0, 0)),
            pl.BlockSpec((1, D_OUT), lambda i: (0, 0)),
        ],
        out_specs=pl.BlockSpec((BM, D_OUT), lambda i: (i, 0)),
        out_shape=jax.ShapeDtypeStruct((NPAD, D_OUT), jnp.float32),
    )(out_raw, stats, gamma.reshape(1, D_OUT), beta.reshape(1, D_OUT))

    return out[:N]


# speedup vs baseline: 15.3849x; 15.3849x over previous
import functools

import jax
import jax.numpy as jnp
from jax import lax
from jax.experimental import pallas as pl
from jax.experimental.pallas import tpu as pltpu
from jax.experimental.pallas import tpu_sc as plsc

N = 10000
E = 320000
D_IN = 128
D_OUT = 256

NPAD = 10240
NC, NS = 2, 16
NW = NC * NS
CHUNK = 128
E2 = E + N
CPT = 81
EPT = CPT * CHUNK
E_PAD = NW * EPT
STRIPE = NPAD // NS
BM = 512
NB = NPAD // BM


def _zero_vmem_2d(ref, nrow, ncol):
    @pl.loop(0, nrow)
    def _(j):
        for g in range(ncol // 16):
            ref[j, pl.ds(g * 16, 16)] = jnp.zeros((16,), jnp.float32)


@functools.cache
def _sc_calls():
    mesh = plsc.VectorSubcoreMesh(
        core_axis_name="c", subcore_axis_name="s", num_cores=NC, num_subcores=NS
    )

    @functools.partial(
        pl.kernel,
        out_type=jax.ShapeDtypeStruct((NC, NPAD), jnp.float32),
        mesh=mesh,
        scratch_types=[
            pltpu.VMEM((CHUNK,), jnp.int32),
            pltpu.VMEM((CHUNK,), jnp.float32),
            pltpu.VMEM((STRIPE,), jnp.float32),
            pltpu.VMEM_SHARED((NPAD,), jnp.float32),
        ],
    )
    def deg_call(dst_hbm, ew_hbm, deg_hbm, dstv, ewv, degv, deg_sh):
        c = lax.axis_index("c")
        s = lax.axis_index("s")
        wid = c * NS + s

        @pl.loop(0, STRIPE // 16)
        def _(i):
            degv[pl.ds(i * 16, 16)] = jnp.zeros((16,), jnp.float32)

        pltpu.sync_copy(degv, deg_sh.at[pl.ds(s * STRIPE, STRIPE)])
        plsc.subcore_barrier()

        base = wid * EPT

        @pl.loop(0, CPT)
        def _(k):
            off = pl.multiple_of(base + k * CHUNK, 8)
            pltpu.sync_copy(dst_hbm.at[pl.ds(off, CHUNK)], dstv)
            pltpu.sync_copy(ew_hbm.at[pl.ds(off, CHUNK)], ewv)
            pltpu.sync_copy(ewv, deg_sh.at[dstv], add=True)

        plsc.subcore_barrier()
        pltpu.sync_copy(deg_sh.at[pl.ds(s * STRIPE, STRIPE)], degv)
        pltpu.sync_copy(degv, deg_hbm.at[c, pl.ds(s * STRIPE, STRIPE)])

    WROWS = NPAD // NW
    XCH = 80

    @functools.partial(
        pl.kernel,
        out_type=jax.ShapeDtypeStruct((NPAD, D_IN), jnp.float32),
        mesh=mesh,
        scratch_types=[
            pltpu.VMEM((WROWS,), jnp.float32),
            pltpu.VMEM((XCH, D_IN), jnp.float32),
        ],
    )
    def xs_call(x_hbm, dis_hbm, xs_hbm, disv, buf):
        c = lax.axis_index("c")
        s = lax.axis_index("s")
        wid = c * NS + s
        r0 = wid * WROWS
        pltpu.sync_copy(dis_hbm.at[pl.ds(r0, WROWS)], disv)
        for t in range(WROWS // XCH):
            rbase = r0 + t * XCH
            pltpu.sync_copy(x_hbm.at[pl.ds(rbase, XCH), :], buf)

            @pl.loop(0, XCH // 16)
            def _(g):
                dv = disv[pl.ds(t * XCH + g * 16, 16)]
                for l in range(16):
                    j = g * 16 + l
                    dj = dv[l]
                    for h in range(D_IN // 16):
                        buf[j, pl.ds(h * 16, 16)] = (
                            buf[j, pl.ds(h * 16, 16)] * dj
                        )

            pltpu.sync_copy(buf, xs_hbm.at[pl.ds(rbase, XCH), :])

    @functools.partial(
        pl.kernel,
        out_type=jax.ShapeDtypeStruct((NC, NPAD, D_IN), jnp.float32),
        mesh=mesh,
        scratch_types=[
            pltpu.VMEM((STRIPE,), jnp.float32),
            pltpu.VMEM((CHUNK,), jnp.int32),
            pltpu.VMEM((CHUNK,), jnp.int32),
            pltpu.VMEM((CHUNK,), jnp.float32),
            pltpu.VMEM((CHUNK, D_IN), jnp.float32),
            pltpu.SemaphoreType.DMA,
            pltpu.VMEM_SHARED((NPAD, D_IN), jnp.float32),
        ],
    )
    def agg_call(src_hbm, dst_hbm, ew_hbm, dis_hbm, xs_hbm, agg_hbm,
                 disv, srcv, dstv, ewv, rows, sem, acc_sh):
        c = lax.axis_index("c")
        s = lax.axis_index("s")
        wid = c * NS + s

        _zero_vmem_2d(rows, CHUNK, D_IN)
        for t in range(STRIPE // CHUNK):
            pltpu.sync_copy(
                rows, acc_sh.at[pl.ds(s * STRIPE + t * CHUNK, CHUNK), :]
            )
        pltpu.sync_copy(dis_hbm.at[pl.ds(s * STRIPE, STRIPE)], disv)
        plsc.subcore_barrier()

        base = wid * EPT

        @pl.loop(0, CPT)
        def _(k):
            off = pl.multiple_of(base + k * CHUNK, 8)
            pltpu.sync_copy(src_hbm.at[pl.ds(off, CHUNK)], srcv)
            pltpu.sync_copy(dst_hbm.at[pl.ds(off, CHUNK)], dstv)
            pltpu.sync_copy(ew_hbm.at[pl.ds(off, CHUNK)], ewv)
            pltpu.async_copy(xs_hbm.at[srcv], rows, sem).wait()

            @pl.loop(0, CHUNK // 16)
            def _(g):
                ev = ewv[pl.ds(g * 16, 16)]
                for l in range(16):
                    j = g * 16 + l
                    ej = ev[l]
                    for h in range(D_IN // 16):
                        rows[j, pl.ds(h * 16, 16)] = (
                            rows[j, pl.ds(h * 16, 16)] * ej
                        )

            pltpu.sync_copy(rows, acc_sh.at[dstv], add=True)

        plsc.subcore_barrier()
        for t in range(STRIPE // CHUNK):
            pltpu.sync_copy(
                acc_sh.at[pl.ds(s * STRIPE + t * CHUNK, CHUNK), :], rows
            )

            @pl.loop(0, CHUNK // 16)
            def _(g):
                dv = disv[pl.ds(t * CHUNK + g * 16, 16)]
                for l in range(16):
                    j = g * 16 + l
                    dj = dv[l]
                    for h in range(D_IN // 16):
                        rows[j, pl.ds(h * 16, 16)] = (
                            rows[j, pl.ds(h * 16, 16)] * dj
                        )

            pltpu.sync_copy(
                rows, agg_hbm.at[c, pl.ds(s * STRIPE + t * CHUNK, CHUNK), :]
            )

    return deg_call, xs_call, agg_call


def _dis_body(degp_ref, dis_ref):
    d = degp_ref[0] + degp_ref[1]
    dis_ref[...] = jnp.where(d > 0, lax.rsqrt(d), 0.0)


def _mm_body(agg_ref, w_ref, b_ref, out_ref, st_ref, s1, s2):
    i = pl.program_id(0)

    @pl.when(i == 0)
    def _():
        s1[...] = jnp.zeros_like(s1)
        s2[...] = jnp.zeros_like(s2)

    a = agg_ref[0] + agg_ref[1]
    o = jnp.dot(a, w_ref[...], preferred_element_type=jnp.float32) + b_ref[...]
    out_ref[...] = o
    rid = i * BM + lax.broadcasted_iota(jnp.int32, (BM, 1), 0)
    m = (rid < N).astype(jnp.float32)
    om = o * m
    s1[...] += jnp.sum(om, axis=0, keepdims=True)
    s2[...] += jnp.sum(om * o, axis=0, keepdims=True)

    @pl.when(i == pl.num_programs(0) - 1)
    def _():
        mean = s1[...] / N
        var = s2[...] / N - (s1[...] / N) * (s1[...] / N)
        st_ref[0:1, :] = mean
        st_ref[1:2, :] = var


def _bn_body(o_ref, st_ref, g_ref, be_ref, out_ref):
    mean = st_ref[0:1, :]
    var = st_ref[1:2, :]
    scale = lax.rsqrt(var + 1e-5) * g_ref[...]
    out_ref[...] = (o_ref[...] - mean) * scale + be_ref[...]


def kernel(x, edge_index, edge_weight, W, b, gamma, beta):
    src = edge_index[0].astype(jnp.int32)
    dst = edge_index[1].astype(jnp.int32)
    ew = edge_weight.astype(jnp.float32)
    loop_idx = jnp.arange(N, dtype=jnp.int32)
    npad_e = E_PAD - E2
    src2 = jnp.concatenate([src, loop_idx, jnp.zeros((npad_e,), jnp.int32)])
    dst2 = jnp.concatenate([dst, loop_idx, jnp.zeros((npad_e,), jnp.int32)])
    ew2 = jnp.concatenate(
        [ew, jnp.ones((N,), jnp.float32), jnp.zeros((npad_e,), jnp.float32)]
    )
    x_pad = jnp.pad(x, ((0, NPAD - N), (0, 0)))

    deg_call, xs_call, agg_call = _sc_calls()
    deg_p = deg_call(dst2, ew2)

    dis = pl.pallas_call(
        _dis_body,
        out_shape=jax.ShapeDtypeStruct((NPAD // 128, 128), jnp.float32),
    )(deg_p.reshape(NC, NPAD // 128, 128)).reshape(NPAD)

    xs = xs_call(x_pad, dis)
    agg_p = agg_call(src2, dst2, ew2, dis, xs)

    out_raw, stats = pl.pallas_call(
        _mm_body,
        grid=(NB,),
        in_specs=[
            pl.BlockSpec((NC, BM, D_IN), lambda i: (0, i, 0)),
            pl.BlockSpec((D_IN, D_OUT), lambda i: (0, 0)),
            pl.BlockSpec((1, D_OUT), lambda i: (0, 0)),
        ],
        out_specs=[
            pl.BlockSpec((BM, D_OUT), lambda i: (i, 0)),
            pl.BlockSpec((2, D_OUT), lambda i: (0, 0)),
        ],
        out_shape=[
            jax.ShapeDtypeStruct((NPAD, D_OUT), jnp.float32),
            jax.ShapeDtypeStruct((2, D_OUT), jnp.float32),
        ],
        scratch_shapes=[
            pltpu.VMEM((1, D_OUT), jnp.float32),
            pltpu.VMEM((1, D_OUT), jnp.float32),
        ],
    )(agg_p, W, b.reshape(1, D_OUT))

    out = pl.pallas_call(
        _bn_body,
        grid=(NB,),
        in_specs=[
            pl.BlockSpec((BM, D_OUT), lambda i: (i, 0)),
            pl.BlockSpec((2, D_OUT), lambda i: (0, 0)),
            pl.BlockSpec((1, D_OUT), lambda i: (0, 0)),
            pl.BlockSpec((1, D_OUT), lambda i: (0, 0)),
        ],
        out_specs=pl.BlockSpec((BM, D_OUT), lambda i: (i, 0)),
        out_shape=jax.ShapeDtypeStruct((NPAD, D_OUT), jnp.float32),
    )(out_raw, stats, gamma.reshape(1, D_OUT), beta.reshape(1, D_OUT))

    return out[:N]


# speedup vs baseline: 28.4273x; 1.8477x over previous
import functools

import jax
import jax.numpy as jnp
from jax import lax
from jax.experimental import pallas as pl
from jax.experimental.pallas import tpu as pltpu
from jax.experimental.pallas import tpu_sc as plsc

N = 10000
E = 320000
D_IN = 128
D_OUT = 256

NPAD = 10240
NC, NS = 2, 16
NW = NC * NS
CHUNK = 128
E2 = E + N
CPT = 81
EPT = CPT * CHUNK
E_PAD = NW * EPT
STRIPE = NPAD // NS
PASSES = 3
CPP = CPT // PASSES
BM = 512
NB = NPAD // BM


def _zero_vmem_2d(ref, nrow, ncol):
    @pl.loop(0, nrow)
    def _(j):
        for g in range(ncol // 16):
            ref[j, pl.ds(g * 16, 16)] = jnp.zeros((16,), jnp.float32)


def _scale_rows_seq(buf, vec_ref, voff, nrow):
    @pl.loop(0, nrow // 16)
    def _(g):
        v = vec_ref[pl.ds(voff + g * 16, 16)]
        for l in range(16):
            j = g * 16 + l
            sj = v[l]
            for h in range(D_IN // 16):
                buf[j, pl.ds(h * 16, 16)] = buf[j, pl.ds(h * 16, 16)] * sj


@functools.cache
def _sc_calls():
    mesh = plsc.VectorSubcoreMesh(
        core_axis_name="c", subcore_axis_name="s", num_cores=NC, num_subcores=NS
    )

    @functools.partial(
        pl.kernel,
        out_type=jax.ShapeDtypeStruct((NC, NPAD), jnp.float32),
        mesh=mesh,
        scratch_types=[
            pltpu.VMEM((CPT, CHUNK), jnp.int32),
            pltpu.VMEM((CPT, CHUNK), jnp.float32),
            pltpu.VMEM((STRIPE,), jnp.float32),
            pltpu.SemaphoreType.DMA,
            pltpu.VMEM_SHARED((NPAD,), jnp.float32),
        ],
    )
    def deg_call(dst_hbm, ew_hbm, deg_hbm, dsta, ewa, degv, sem, deg_sh):
        c = lax.axis_index("c")
        s = lax.axis_index("s")
        wid = c * NS + s

        @pl.loop(0, STRIPE // 16)
        def _(i):
            degv[pl.ds(i * 16, 16)] = jnp.zeros((16,), jnp.float32)

        pltpu.sync_copy(degv, deg_sh.at[pl.ds(s * STRIPE, STRIPE)])
        pltpu.sync_copy(dst_hbm.at[wid], dsta)
        pltpu.sync_copy(ew_hbm.at[wid], ewa)
        plsc.subcore_barrier()

        @pl.loop(0, CPT)
        def _(k):
            pltpu.async_copy(ewa.at[k], deg_sh.at[dsta.at[k]], sem, add=True)

        pltpu.make_async_copy(ew_hbm.at[wid], ewa, sem).wait()
        plsc.subcore_barrier()
        pltpu.sync_copy(deg_sh.at[pl.ds(s * STRIPE, STRIPE)], degv)
        pltpu.sync_copy(degv, deg_hbm.at[c, pl.ds(s * STRIPE, STRIPE)])

    WROWS = NPAD // NW
    XCH = 80

    @functools.partial(
        pl.kernel,
        out_type=jax.ShapeDtypeStruct((NPAD, D_IN), jnp.float32),
        mesh=mesh,
        scratch_types=[
            pltpu.VMEM((WROWS,), jnp.float32),
            pltpu.VMEM((XCH, D_IN), jnp.float32),
        ],
    )
    def xs_call(x_hbm, dis_hbm, xs_hbm, disv, buf):
        c = lax.axis_index("c")
        s = lax.axis_index("s")
        wid = c * NS + s
        r0 = wid * WROWS
        pltpu.sync_copy(dis_hbm.at[pl.ds(r0, WROWS)], disv)
        for t in range(WROWS // XCH):
            rbase = r0 + t * XCH
            pltpu.sync_copy(x_hbm.at[pl.ds(rbase, XCH), :], buf)
            _scale_rows_seq(buf, disv, t * XCH, XCH)
            pltpu.sync_copy(buf, xs_hbm.at[pl.ds(rbase, XCH), :])

    @functools.partial(
        pl.kernel,
        out_type=jax.ShapeDtypeStruct((NC, NPAD, D_IN), jnp.float32),
        mesh=mesh,
        scratch_types=[
            pltpu.VMEM((STRIPE,), jnp.float32),
            pltpu.VMEM((CPP, 2, CHUNK), jnp.int32),
            pltpu.VMEM((CPP, CHUNK), jnp.float32),
            pltpu.VMEM((CHUNK, D_IN), jnp.float32),
            pltpu.VMEM((CHUNK, D_IN), jnp.float32),
            pltpu.SemaphoreType.DMA,
            pltpu.SemaphoreType.DMA,
            pltpu.SemaphoreType.DMA,
            pltpu.SemaphoreType.DMA,
            pltpu.VMEM_SHARED((NPAD, D_IN), jnp.float32),
        ],
    )
    def agg_call(edges_hbm, ew_hbm, dis_hbm, xs_hbm, agg_hbm,
                 disv, ebuf, ewb, r0b, r1b, g0, g1, s0, s1, acc_sh):
        c = lax.axis_index("c")
        s = lax.axis_index("s")
        wid = c * NS + s
        rows = (r0b, r1b)
        gsem = (g0, g1)
        ssem = (s0, s1)

        _zero_vmem_2d(r0b, CHUNK, D_IN)
        for t in range(STRIPE // CHUNK):
            pltpu.sync_copy(
                r0b, acc_sh.at[pl.ds(s * STRIPE + t * CHUNK, CHUNK), :]
            )
        pltpu.sync_copy(dis_hbm.at[pl.ds(s * STRIPE, STRIPE)], disv)
        plsc.subcore_barrier()

        def gath(k, b):
            pltpu.async_copy(xs_hbm.at[ebuf.at[k, 0]], rows[b], gsem[b])

        def wait_g(b):
            pltpu.make_async_copy(
                xs_hbm.at[ebuf.at[0, 0]], rows[b], gsem[b]
            ).wait()

        def scat(k, b):
            pltpu.async_copy(rows[b], acc_sh.at[ebuf.at[k, 1]], ssem[b],
                             add=True)

        def wait_s(b):
            pltpu.make_async_copy(
                rows[b], acc_sh.at[ebuf.at[0, 1]], ssem[b]
            ).wait()

        def scale(k, b):
            buf = rows[b]

            @pl.loop(0, CHUNK // 16)
            def _(g):
                ev = ewb[k, pl.ds(g * 16, 16)]
                for l in range(16):
                    j = g * 16 + l
                    ej = ev[l]
                    for h in range(D_IN // 16):
                        buf[j, pl.ds(h * 16, 16)] = (
                            buf[j, pl.ds(h * 16, 16)] * ej
                        )

        for p in range(PASSES):
            pltpu.sync_copy(edges_hbm.at[wid, p], ebuf)
            pltpu.sync_copy(ew_hbm.at[wid, p], ewb)
            gath(0, 0)
            wait_g(0)
            scale(0, 0)
            scat(0, 0)
            gath(1, 1)

            @pl.loop(0, (CPP - 3) // 2)
            def _(r):
                k1 = 2 * r + 1
                wait_g(1)
                scale(k1, 1)
                scat(k1, 1)
                wait_s(0)
                gath(k1 + 1, 0)
                wait_g(0)
                scale(k1 + 1, 0)
                scat(k1 + 1, 0)
                wait_s(1)
                gath(k1 + 2, 1)

            wait_g(1)
            scale(CPP - 2, 1)
            scat(CPP - 2, 1)
            wait_s(0)
            gath(CPP - 1, 0)
            wait_g(0)
            scale(CPP - 1, 0)
            scat(CPP - 1, 0)
            wait_s(1)
            wait_s(0)

        plsc.subcore_barrier()
        for t in range(STRIPE // CHUNK):
            pltpu.sync_copy(
                acc_sh.at[pl.ds(s * STRIPE + t * CHUNK, CHUNK), :], r0b
            )
            _scale_rows_seq(r0b, disv, t * CHUNK, CHUNK)
            pltpu.sync_copy(
                r0b, agg_hbm.at[c, pl.ds(s * STRIPE + t * CHUNK, CHUNK), :]
            )

    return deg_call, xs_call, agg_call


def _dis_body(degp_ref, dis_ref):
    d = degp_ref[0] + degp_ref[1]
    dis_ref[...] = jnp.where(d > 0, lax.rsqrt(d), 0.0)


def _mm_bn_body(agg_ref, w_ref, b_ref, g_ref, be_ref, out_ref, s1, s2):
    p = pl.program_id(0)
    i = pl.program_id(1)

    @pl.when((p == 0) & (i == 0))
    def _():
        s1[...] = jnp.zeros_like(s1)
        s2[...] = jnp.zeros_like(s2)

    a = agg_ref[0] + agg_ref[1]
    o = jnp.dot(a, w_ref[...], preferred_element_type=jnp.float32) + b_ref[...]

    @pl.when(p == 0)
    def _():
        rid = i * BM + lax.broadcasted_iota(jnp.int32, (BM, 1), 0)
        m = (rid < N).astype(jnp.float32)
        om = o * m
        s1[...] += jnp.sum(om, axis=0, keepdims=True)
        s2[...] += jnp.sum(om * o, axis=0, keepdims=True)

    @pl.when(p == 1)
    def _():
        mean = s1[...] / N
        var = s2[...] / N - mean * mean
        scale = lax.rsqrt(var + 1e-5) * g_ref[...]
        out_ref[...] = (o - mean) * scale + be_ref[...]


def kernel(x, edge_index, edge_weight, W, b, gamma, beta):
    src = edge_index[0].astype(jnp.int32)
    dst = edge_index[1].astype(jnp.int32)
    ew = edge_weight.astype(jnp.float32)
    loop_idx = jnp.arange(N, dtype=jnp.int32)
    npad_e = E_PAD - E2
    pad_idx = (jnp.arange(npad_e, dtype=jnp.int32)) % N
    src2 = jnp.concatenate([src, loop_idx, pad_idx]).reshape(NW, CPT, CHUNK)
    dst2 = jnp.concatenate([dst, loop_idx, pad_idx]).reshape(NW, CPT, CHUNK)
    ew2 = jnp.concatenate(
        [ew, jnp.ones((N,), jnp.float32), jnp.zeros((npad_e,), jnp.float32)]
    ).reshape(NW, CPT, CHUNK)
    edges = jnp.stack([src2, dst2], axis=2).reshape(
        NW, PASSES, CPP, 2, CHUNK
    )
    ew3 = ew2.reshape(NW, PASSES, CPP, CHUNK)
    x_pad = jnp.pad(x, ((0, NPAD - N), (0, 0)))

    deg_call, xs_call, agg_call = _sc_calls()
    deg_p = deg_call(dst2, ew2)

    dis = pl.pallas_call(
        _dis_body,
        out_shape=jax.ShapeDtypeStruct((NPAD // 128, 128), jnp.float32),
    )(deg_p.reshape(NC, NPAD // 128, 128)).reshape(NPAD)

    xs = xs_call(x_pad, dis)
    agg_p = agg_call(edges, ew3, dis, xs)

    out = pl.pallas_call(
        _mm_bn_body,
        grid=(2, NB),
        in_specs=[
            pl.BlockSpec((NC, BM, D_IN), lambda p, i: (0, i, 0)),
            pl.BlockSpec((D_IN, D_OUT), lambda p, i: (0, 0)),
            pl.BlockSpec((1, D_OUT), lambda p, i: (0, 0)),
            pl.BlockSpec((1, D_OUT), lambda p, i: (0, 0)),
            pl.BlockSpec((1, D_OUT), lambda p, i: (0, 0)),
        ],
        out_specs=pl.BlockSpec((BM, D_OUT), lambda p, i: (i, 0)),
        out_shape=jax.ShapeDtypeStruct((NPAD, D_OUT), jnp.float32),
        scratch_shapes=[
            pltpu.VMEM((1, D_OUT), jnp.float32),
            pltpu.VMEM((1, D_OUT), jnp.float32),
        ],
    )(agg_p, W, b.reshape(1, D_OUT), gamma.reshape(1, D_OUT),
      beta.reshape(1, D_OUT))

    return out[:N]


# speedup vs baseline: 28.8224x; 1.0139x over previous
import functools

import jax
import jax.numpy as jnp
from jax import lax
from jax.experimental import pallas as pl
from jax.experimental.pallas import tpu as pltpu
from jax.experimental.pallas import tpu_sc as plsc

N = 10000
E = 320000
D_IN = 128
D_OUT = 256

NPAD = 10240
NC, NS = 2, 16
NW = NC * NS
CHUNK = 128
E2 = E + N
CPT = 81
EPT = CPT * CHUNK
E_PAD = NW * EPT
STRIPE = NPAD // NS
PASSES = 3
CPP = CPT // PASSES
BM = 512
NB = NPAD // BM


def _zero_vmem_2d(ref, nrow, ncol):
    @pl.loop(0, nrow)
    def _(j):
        for g in range(ncol // 16):
            ref[j, pl.ds(g * 16, 16)] = jnp.zeros((16,), jnp.float32)


def _scale_rows_seq(buf, vec_ref, voff, nrow):
    @pl.loop(0, nrow // 16)
    def _(g):
        v = vec_ref[pl.ds(voff + g * 16, 16)]
        for l in range(16):
            j = g * 16 + l
            sj = v[l]
            for h in range(D_IN // 16):
                buf[j, pl.ds(h * 16, 16)] = buf[j, pl.ds(h * 16, 16)] * sj


@functools.cache
def _sc_calls():
    mesh = plsc.VectorSubcoreMesh(
        core_axis_name="c", subcore_axis_name="s", num_cores=NC, num_subcores=NS
    )

    @functools.partial(
        pl.kernel,
        out_type=jax.ShapeDtypeStruct((NC, NPAD), jnp.float32),
        mesh=mesh,
        scratch_types=[
            pltpu.VMEM((CPT, CHUNK), jnp.int32),
            pltpu.VMEM((CPT, CHUNK), jnp.float32),
            pltpu.VMEM((STRIPE,), jnp.float32),
            pltpu.SemaphoreType.DMA,
            pltpu.VMEM_SHARED((NPAD,), jnp.float32),
        ],
    )
    def deg_call(dst_hbm, ew_hbm, deg_hbm, dsta, ewa, degv, sem, deg_sh):
        c = lax.axis_index("c")
        s = lax.axis_index("s")
        wid = c * NS + s

        @pl.loop(0, STRIPE // 16)
        def _(i):
            degv[pl.ds(i * 16, 16)] = jnp.zeros((16,), jnp.float32)

        pltpu.sync_copy(degv, deg_sh.at[pl.ds(s * STRIPE, STRIPE)])
        pltpu.sync_copy(dst_hbm.at[wid], dsta)
        pltpu.sync_copy(ew_hbm.at[wid], ewa)
        plsc.subcore_barrier()

        @pl.loop(0, CPT)
        def _(k):
            pltpu.async_copy(ewa.at[k], deg_sh.at[dsta.at[k]], sem, add=True)

        pltpu.make_async_copy(ew_hbm.at[wid], ewa, sem).wait()
        plsc.subcore_barrier()
        pltpu.sync_copy(deg_sh.at[pl.ds(s * STRIPE, STRIPE)], degv)
        pltpu.sync_copy(degv, deg_hbm.at[c, pl.ds(s * STRIPE, STRIPE)])

    WROWS = NPAD // NW
    XCH = 80

    @functools.partial(
        pl.kernel,
        out_type=jax.ShapeDtypeStruct((NPAD, D_IN), jnp.float32),
        mesh=mesh,
        scratch_types=[
            pltpu.VMEM((WROWS,), jnp.float32),
            pltpu.VMEM((XCH, D_IN), jnp.float32),
        ],
    )
    def xs_call(x_hbm, dis_hbm, xs_hbm, disv, buf):
        c = lax.axis_index("c")
        s = lax.axis_index("s")
        wid = c * NS + s
        r0 = wid * WROWS
        pltpu.sync_copy(dis_hbm.at[pl.ds(r0, WROWS)], disv)

        for t in range(WROWS // XCH):
            rbase = r0 + t * XCH
            pltpu.sync_copy(x_hbm.at[pl.ds(rbase, XCH), :], buf)
            _scale_rows_seq(buf, disv, t * XCH, XCH)
            pltpu.sync_copy(buf, xs_hbm.at[pl.ds(rbase, XCH), :])

    @functools.partial(
        pl.kernel,
        out_type=jax.ShapeDtypeStruct((NC, NPAD, D_IN), jnp.float32),
        mesh=mesh,
        scratch_types=[
            pltpu.VMEM((STRIPE,), jnp.float32),
            pltpu.VMEM((CPP, 2, CHUNK), jnp.int32),
            pltpu.VMEM((CPP, CHUNK), jnp.float32),
            pltpu.VMEM((CHUNK, D_IN), jnp.float32),
            pltpu.VMEM((CHUNK, D_IN), jnp.float32),
            pltpu.SemaphoreType.DMA,
            pltpu.SemaphoreType.DMA,
            pltpu.SemaphoreType.DMA,
            pltpu.SemaphoreType.DMA,
            pltpu.VMEM_SHARED((NPAD, D_IN), jnp.float32),
        ],
    )
    def agg_call(edges_hbm, ew_hbm, dis_hbm, xs_hbm, agg_hbm,
                 disv, ebuf, ewb, r0b, r1b, g0, g1, s0, s1, acc_sh):
        c = lax.axis_index("c")
        s = lax.axis_index("s")
        wid = c * NS + s
        rows = (r0b, r1b)
        gsem = (g0, g1)
        ssem = (s0, s1)

        _zero_vmem_2d(r0b, CHUNK, D_IN)
        for t in range(STRIPE // CHUNK):
            pltpu.sync_copy(
                r0b, acc_sh.at[pl.ds(s * STRIPE + t * CHUNK, CHUNK), :]
            )
        pltpu.sync_copy(dis_hbm.at[pl.ds(s * STRIPE, STRIPE)], disv)
        plsc.subcore_barrier()

        def gath(k, b):
            pltpu.async_copy(xs_hbm.at[ebuf.at[k, 0]], rows[b], gsem[b])

        def wait_g(b):
            pltpu.make_async_copy(
                xs_hbm.at[ebuf.at[0, 0]], rows[b], gsem[b]
            ).wait()

        def scat(k, b):
            pltpu.async_copy(rows[b], acc_sh.at[ebuf.at[k, 1]], ssem[b],
                             add=True)

        def wait_s(b):
            pltpu.make_async_copy(
                rows[b], acc_sh.at[ebuf.at[0, 1]], ssem[b]
            ).wait()

        def scale(k, b):
            buf = rows[b]

            @pl.loop(0, CHUNK // 16)
            def _(g):
                ev = ewb[k, pl.ds(g * 16, 16)]
                for l in range(16):
                    j = g * 16 + l
                    ej = ev[l]
                    for h in range(D_IN // 16):
                        buf[j, pl.ds(h * 16, 16)] = (
                            buf[j, pl.ds(h * 16, 16)] * ej
                        )

        @pl.loop(0, PASSES)
        def _(p):
            pltpu.sync_copy(edges_hbm.at[wid, p], ebuf)
            pltpu.sync_copy(ew_hbm.at[wid, p], ewb)
            gath(0, 0)
            wait_g(0)
            scale(0, 0)
            scat(0, 0)
            gath(1, 1)

            @pl.loop(0, (CPP - 3) // 2)
            def _(r):
                k1 = 2 * r + 1
                wait_g(1)
                scale(k1, 1)
                scat(k1, 1)
                wait_s(0)
                gath(k1 + 1, 0)
                wait_g(0)
                scale(k1 + 1, 0)
                scat(k1 + 1, 0)
                wait_s(1)
                gath(k1 + 2, 1)

            wait_g(1)
            scale(CPP - 2, 1)
            scat(CPP - 2, 1)
            wait_s(0)
            gath(CPP - 1, 0)
            wait_g(0)
            scale(CPP - 1, 0)
            scat(CPP - 1, 0)
            wait_s(1)
            wait_s(0)

        plsc.subcore_barrier()
        for t in range(STRIPE // CHUNK):
            pltpu.sync_copy(
                acc_sh.at[pl.ds(s * STRIPE + t * CHUNK, CHUNK), :], r0b
            )
            _scale_rows_seq(r0b, disv, t * CHUNK, CHUNK)
            pltpu.sync_copy(
                r0b, agg_hbm.at[c, pl.ds(s * STRIPE + t * CHUNK, CHUNK), :]
            )

    return deg_call, xs_call, agg_call


def _dis_body(degp_ref, dis_ref):
    d = degp_ref[0] + degp_ref[1]
    dis_ref[...] = jnp.where(d > 0, lax.rsqrt(d), 0.0)


def _mm_bn_body(agg_ref, w_ref, b_ref, g_ref, be_ref, out_ref, s1, s2):
    p = pl.program_id(0)
    i = pl.program_id(1)

    @pl.when((p == 0) & (i == 0))
    def _():
        s1[...] = jnp.zeros_like(s1)
        s2[...] = jnp.zeros_like(s2)

    a = agg_ref[0] + agg_ref[1]
    o = jnp.dot(a, w_ref[...], preferred_element_type=jnp.float32) + b_ref[...]

    @pl.when(p == 0)
    def _():
        rid = i * BM + lax.broadcasted_iota(jnp.int32, (BM, 1), 0)
        m = (rid < N).astype(jnp.float32)
        om = o * m
        s1[...] += jnp.sum(om, axis=0, keepdims=True)
        s2[...] += jnp.sum(om * o, axis=0, keepdims=True)

    @pl.when(p == 1)
    def _():
        mean = s1[...] / N
        var = s2[...] / N - mean * mean
        scale = lax.rsqrt(var + 1e-5) * g_ref[...]
        out_ref[...] = (o - mean) * scale + be_ref[...]


def kernel(x, edge_index, edge_weight, W, b, gamma, beta):
    src = edge_index[0].astype(jnp.int32)
    dst = edge_index[1].astype(jnp.int32)
    ew = edge_weight.astype(jnp.float32)
    loop_idx = jnp.arange(N, dtype=jnp.int32)
    npad_e = E_PAD - E2
    pad_idx = (jnp.arange(npad_e, dtype=jnp.int32)) % N
    src2 = jnp.concatenate([src, loop_idx, pad_idx]).reshape(NW, CPT, CHUNK)
    dst2 = jnp.concatenate([dst, loop_idx, pad_idx]).reshape(NW, CPT, CHUNK)
    ew2 = jnp.concatenate(
        [ew, jnp.ones((N,), jnp.float32), jnp.zeros((npad_e,), jnp.float32)]
    ).reshape(NW, CPT, CHUNK)
    edges = jnp.stack([src2, dst2], axis=2).reshape(NW, PASSES, CPP, 2, CHUNK)
    ew3 = ew2.reshape(NW, PASSES, CPP, CHUNK)
    x_pad = jnp.pad(x, ((0, NPAD - N), (0, 0)))

    deg_call, xs_call, agg_call = _sc_calls()
    deg_p = deg_call(dst2, ew2)

    dis = pl.pallas_call(
        _dis_body,
        out_shape=jax.ShapeDtypeStruct((NPAD // 128, 128), jnp.float32),
    )(deg_p.reshape(NC, NPAD // 128, 128)).reshape(NPAD)

    xs = xs_call(x_pad, dis)
    agg_p = agg_call(edges, ew3, dis, xs)

    out = pl.pallas_call(
        _mm_bn_body,
        grid=(2, NB),
        in_specs=[
            pl.BlockSpec((NC, BM, D_IN), lambda p, i: (0, i, 0)),
            pl.BlockSpec((D_IN, D_OUT), lambda p, i: (0, 0)),
            pl.BlockSpec((1, D_OUT), lambda p, i: (0, 0)),
            pl.BlockSpec((1, D_OUT), lambda p, i: (0, 0)),
            pl.BlockSpec((1, D_OUT), lambda p, i: (0, 0)),
        ],
        out_specs=pl.BlockSpec((BM, D_OUT), lambda p, i: (i, 0)),
        out_shape=jax.ShapeDtypeStruct((NPAD, D_OUT), jnp.float32),
        scratch_shapes=[
            pltpu.VMEM((1, D_OUT), jnp.float32),
            pltpu.VMEM((1, D_OUT), jnp.float32),
        ],
    )(agg_p, W, b.reshape(1, D_OUT), gamma.reshape(1, D_OUT),
      beta.reshape(1, D_OUT))

    return out[:N]
